# baseline (speedup 1.0000x reference)
import functools

import jax
import jax.numpy as jnp
from jax import lax
from jax.experimental import pallas as pl
from jax.experimental.pallas import tpu as pltpu
from jax.experimental.pallas import tpu_sc as plsc

NUM_ITEM = 1000000
HIDDEN = 64
SEQ = 200
BATCH = 4096
PAD = 3

NC, NS, L = 2, 16, 16
NW = NC * NS
N = BATCH * SEQ
PER_W = N // NW
C = 800
G = PER_W // C
PIECES = [(o, min(128, C - o)) for o in range(0, C, 128)]


def _body(x_hbm, tbl_hbm, aux_hbm, posm_hbm, out_hbm, mask_hbm,
          idx_v0, auxi_v0, mask_v0, dest_v0,
          idx_v1, auxi_v1, mask_v1, dest_v1,
          posm_v, sem_g0, sem_a0, sem_o0, sem_g1, sem_a1, sem_o1):
    wid = lax.axis_index("s") * NC + lax.axis_index("c")
    w0 = wid * PER_W

    pltpu.sync_copy(posm_hbm, posm_v)

    bufs = [(idx_v0, auxi_v0, mask_v0, dest_v0, sem_g0, sem_a0, sem_o0),
            (idx_v1, auxi_v1, mask_v1, dest_v1, sem_g1, sem_a1, sem_o1)]

    def drain_out(b):
        _, _, mask_v, dest_v, _, _, sem_o = bufs[b]
        pltpu.make_async_copy(dest_v, out_hbm.at[pl.ds(0, C)], sem_o).wait()
        pltpu.make_async_copy(mask_v, mask_hbm.at[pl.ds(0, C)], sem_o).wait()

    def stage1(g, b):
        idx_v, auxi_v, mask_v, dest_v, sem_g, _, _ = bufs[b]
        base = w0 + g * C
        pltpu.sync_copy(x_hbm.at[pl.ds(base, C)], idx_v)
        for j in range(C // L):
            sl = pl.ds(j * L, L)
            iv = idx_v[sl]
            pad = iv == PAD
            auxi_v[sl] = jnp.where(pad, SEQ, posm_v[sl])
            mask_v[sl] = jnp.where(pad, 1, 0)
        return [
            pltpu.async_copy(aux_hbm.at[auxi_v.at[pl.ds(o, s)]],
                             dest_v.at[pl.ds(o, s)], sem_g)
            for o, s in PIECES
        ]

    def fire_add(b):
        idx_v, _, _, dest_v, _, sem_a, _ = bufs[b]
        return [
            pltpu.async_copy(tbl_hbm.at[idx_v.at[pl.ds(o, s)]],
                             dest_v.at[pl.ds(o, s)], sem_a, add=True)
            for o, s in PIECES
        ]

    def fire_out(g, b):
        _, _, mask_v, dest_v, _, _, sem_o = bufs[b]
        base = w0 + g * C
        pltpu.async_copy(dest_v, out_hbm.at[pl.ds(base, C)], sem_o)
        pltpu.async_copy(mask_v, mask_hbm.at[pl.ds(base, C)], sem_o)

    @pl.loop(0, G, step=2)
    def _chunk(g):
        @pl.when(g >= 2)
        def _():
            drain_out(0)

        pre_a = stage1(g, 0)

        @pl.when(g >= 2)
        def _():
            drain_out(1)

        pre_b = stage1(g + 1, 1)
        for d in pre_a:
            d.wait()
        add_a = fire_add(0)
        for d in add_a:
            d.wait()
        fire_out(g, 0)
        for d in pre_b:
            d.wait()
        add_b = fire_add(1)
        for d in add_b:
            d.wait()
        fire_out(g + 1, 1)

    drain_out(0)
    drain_out(1)


@jax.jit
def _sc_embed(xf, item_table, aux, posm):
    return pl.kernel(
        _body,
        out_type=[
            jax.ShapeDtypeStruct((N, HIDDEN), jnp.float32),
            jax.ShapeDtypeStruct((N,), jnp.int32),
        ],
        mesh=plsc.VectorSubcoreMesh(
            core_axis_name="c", subcore_axis_name="s",
            num_cores=NC, num_subcores=NS),
        compiler_params=pltpu.CompilerParams(use_tc_tiling_on_sc=False),
        scratch_types=(
            [pltpu.VMEM((C,), jnp.int32),
             pltpu.VMEM((C,), jnp.int32),
             pltpu.VMEM((C,), jnp.int32),
             pltpu.VMEM((C, HIDDEN), jnp.float32)] * 2
            + [pltpu.VMEM((C,), jnp.int32)]
            + [pltpu.SemaphoreType.DMA] * 6
        ),
    )(xf, item_table, aux, posm)


def kernel(x, item_table, pos_table):
    xf = x.reshape(N)
    aux = jnp.concatenate([pos_table, -item_table[PAD:PAD + 1]], axis=0)
    posm = jnp.tile(jnp.arange(SEQ, dtype=jnp.int32), C // SEQ)
    emb, mask = _sc_embed(xf, item_table, aux, posm)
    return (emb.reshape(BATCH, SEQ, HIDDEN),
            mask.reshape(BATCH, SEQ).astype(bool))


# speedup vs baseline: 1.0263x; 1.0263x over previous
import functools

import jax
import jax.numpy as jnp
from jax import lax
from jax.experimental import pallas as pl
from jax.experimental.pallas import tpu as pltpu
from jax.experimental.pallas import tpu_sc as plsc

NUM_ITEM = 1000000
HIDDEN = 64
W = 128
SEQ = 200
BATCH = 4096
PAD = 3

NC, NS, L = 2, 16, 16
NW = NC * NS
N = BATCH * SEQ
PER_W = N // NW
C = 400
G = PER_W // C
PIECES = [(o, min(128, C - o)) for o in range(0, C, 128)]


def _body(x_hbm, tbl_hbm, aux_hbm, posm_hbm, out_hbm, mask_hbm,
          idx_v0, auxi_v0, mask_v0, dest_v0,
          idx_v1, auxi_v1, mask_v1, dest_v1,
          posm_v, sem_g0, sem_a0, sem_o0, sem_g1, sem_a1, sem_o1):
    wid = lax.axis_index("s") * NC + lax.axis_index("c")
    w0 = wid * PER_W

    pltpu.sync_copy(posm_hbm, posm_v)

    bufs = [(idx_v0, auxi_v0, mask_v0, dest_v0, sem_g0, sem_a0, sem_o0),
            (idx_v1, auxi_v1, mask_v1, dest_v1, sem_g1, sem_a1, sem_o1)]

    def drain_out(b):
        _, _, mask_v, dest_v, _, _, sem_o = bufs[b]
        pltpu.make_async_copy(dest_v, out_hbm.at[pl.ds(0, C)], sem_o).wait()
        pltpu.make_async_copy(mask_v, mask_hbm.at[pl.ds(0, C)], sem_o).wait()

    def stage1(g, b):
        idx_v, auxi_v, mask_v, dest_v, sem_g, _, _ = bufs[b]
        base = w0 + g * C
        pltpu.sync_copy(x_hbm.at[pl.ds(base, C)], idx_v)
        for j in range(C // L):
            sl = pl.ds(j * L, L)
            iv = idx_v[sl]
            pad = iv == PAD
            auxi_v[sl] = jnp.where(pad, SEQ, posm_v[sl])
            mask_v[sl] = jnp.where(pad, 1, 0)
        return [
            pltpu.async_copy(aux_hbm.at[auxi_v.at[pl.ds(o, s)]],
                             dest_v.at[pl.ds(o, s)], sem_g)
            for o, s in PIECES
        ]

    def fire_add(b):
        idx_v, _, _, dest_v, _, sem_a, _ = bufs[b]
        return [
            pltpu.async_copy(tbl_hbm.at[idx_v.at[pl.ds(o, s)]],
                             dest_v.at[pl.ds(o, s)], sem_a, add=True)
            for o, s in PIECES
        ]

    def fire_out(g, b):
        _, _, mask_v, dest_v, _, _, sem_o = bufs[b]
        base = w0 + g * C
        pltpu.async_copy(dest_v, out_hbm.at[pl.ds(base, C)], sem_o)
        pltpu.async_copy(mask_v, mask_hbm.at[pl.ds(base, C)], sem_o)

    @pl.loop(0, G, step=2)
    def _chunk(g):
        @pl.when(g >= 2)
        def _():
            drain_out(0)

        pre_a = stage1(g, 0)

        @pl.when(g >= 2)
        def _():
            drain_out(1)

        pre_b = stage1(g + 1, 1)
        for d in pre_a:
            d.wait()
        add_a = fire_add(0)
        for d in add_a:
            d.wait()
        fire_out(g, 0)
        for d in pre_b:
            d.wait()
        add_b = fire_add(1)
        for d in add_b:
            d.wait()
        fire_out(g + 1, 1)

    drain_out(0)
    drain_out(1)


@jax.jit
def _sc_embed(xf, tblp, auxp, posm):
    return pl.kernel(
        _body,
        out_type=[
            jax.ShapeDtypeStruct((N, W), jnp.float32),
            jax.ShapeDtypeStruct((N,), jnp.int32),
        ],
        mesh=plsc.VectorSubcoreMesh(
            core_axis_name="c", subcore_axis_name="s",
            num_cores=NC, num_subcores=NS),
        compiler_params=pltpu.CompilerParams(use_tc_tiling_on_sc=True),
        scratch_types=(
            [pltpu.VMEM((C,), jnp.int32),
             pltpu.VMEM((C,), jnp.int32),
             pltpu.VMEM((C,), jnp.int32),
             pltpu.VMEM((C, W), jnp.float32)] * 2
            + [pltpu.VMEM((C,), jnp.int32)]
            + [pltpu.SemaphoreType.DMA] * 6
        ),
    )(xf, tblp, auxp, posm)


def kernel(x, item_table, pos_table):
    xf = x.reshape(N)
    tblp = jnp.pad(item_table, ((0, 0), (0, W - HIDDEN)))
    aux = jnp.concatenate([pos_table, -item_table[PAD:PAD + 1]], axis=0)
    auxp = jnp.pad(aux, ((0, 0), (0, W - HIDDEN)))
    posm = jnp.tile(jnp.arange(SEQ, dtype=jnp.int32), C // SEQ)
    emb, mask = _sc_embed(xf, tblp, auxp, posm)
    return (emb[:, :HIDDEN].reshape(BATCH, SEQ, HIDDEN),
            mask.reshape(BATCH, SEQ).astype(bool))


# speedup vs baseline: 1.5158x; 1.4770x over previous
import functools

import jax
import jax.numpy as jnp
from jax import lax
from jax.experimental import pallas as pl
from jax.experimental.pallas import tpu as pltpu
from jax.experimental.pallas import tpu_sc as plsc

NUM_ITEM = 1000000
HIDDEN = 64
W = 128
SEQ = 200
BATCH = 4096
PAD = 3

NC, NS, L = 2, 16, 16
NW = NC * NS
N = BATCH * SEQ
PER_W = N // NW
C = 400
G = PER_W // C
PIECES = [(o, min(128, C - o)) for o in range(0, C, 128)]
NREP = C // SEQ


def _body(x_hbm, tbl_hbm, pos_hbm, out_hbm, mask_hbm,
          idx_v0, mask_v0, padl_v0, dest_v0,
          idx_v1, mask_v1, padl_v1, dest_v1,
          pos_v, sem_a0, sem_o0, sem_a1, sem_o1):
    wid = lax.axis_index("s") * NC + lax.axis_index("c")
    w0 = wid * PER_W
    lanes = jax.lax.iota(jnp.int32, L)

    pltpu.sync_copy(pos_hbm, pos_v)

    bufs = [(idx_v0, mask_v0, padl_v0, dest_v0, sem_a0, sem_o0),
            (idx_v1, mask_v1, padl_v1, dest_v1, sem_a1, sem_o1)]

    def drain_out(b):
        _, mask_v, _, dest_v, _, sem_o = bufs[b]
        pltpu.make_async_copy(dest_v.at[pl.ds(0, C)],
                              out_hbm.at[pl.ds(0, C)], sem_o).wait()
        pltpu.make_async_copy(mask_v, mask_hbm.at[pl.ds(0, C)], sem_o).wait()

    def stage1(g, b):
        idx_v, mask_v, padl_v, dest_v, sem_a, _ = bufs[b]
        base = w0 + g * C
        pltpu.sync_copy(x_hbm.at[pl.ds(base, C)], idx_v)
        cnt = jnp.int32(0)
        for j in range(C // L):
            sl = pl.ds(j * L, L)
            pad = idx_v[sl] == PAD
            padi = jnp.where(pad, 1, 0)
            mask_v[sl] = padi
            cum = plsc.cumsum(padi)
            tgt = jnp.where(pad, cnt + cum - 1, C + L)
            plsc.store_scatter(padl_v, [tgt], lanes + (j * L))
            cnt = cnt + jnp.max(cum)
        padl_v[pl.ds(cnt, L)] = jnp.full((L,), C, jnp.int32)
        for s in range(SEQ):
            for c in range(HIDDEN // L):
                v = pos_v[pl.ds(s * HIDDEN + c * L, L)]
                for rep in range(NREP):
                    dest_v[s + rep * SEQ, pl.ds(c * L, L)] = v
        descs = [
            pltpu.async_copy(tbl_hbm.at[idx_v.at[pl.ds(o, sz)]],
                             dest_v.at[pl.ds(o, sz)], sem_a, add=True)
            for o, sz in PIECES
        ]
        return descs, cnt

    def finish(g, b, descs, cnt):
        _, mask_v, padl_v, dest_v, _, sem_o = bufs[b]
        for d in descs:
            d.wait()

        zeros = jnp.zeros((L,), jnp.float32)

        @pl.loop(0, (cnt + L - 1) // L)
        def _fix(t):
            rows = padl_v[pl.ds(t * L, L)]
            for k in range(HIDDEN):
                plsc.store_scatter(
                    dest_v, [rows, jnp.full((L,), k, jnp.int32)], zeros)

        base = w0 + g * C
        pltpu.async_copy(dest_v.at[pl.ds(0, C)],
                         out_hbm.at[pl.ds(base, C)], sem_o)
        pltpu.async_copy(mask_v, mask_hbm.at[pl.ds(base, C)], sem_o)

    @pl.loop(0, G, step=2)
    def _chunk(g):
        @pl.when(g >= 2)
        def _():
            drain_out(0)

        da, ca = stage1(g, 0)

        @pl.when(g >= 2)
        def _():
            drain_out(1)

        db, cb = stage1(g + 1, 1)
        finish(g, 0, da, ca)
        finish(g + 1, 1, db, cb)

    drain_out(0)
    drain_out(1)


@jax.jit
def _sc_embed(xf, tblp, posf):
    return pl.kernel(
        _body,
        out_type=[
            jax.ShapeDtypeStruct((N, W), jnp.float32),
            jax.ShapeDtypeStruct((N,), jnp.int32),
        ],
        mesh=plsc.VectorSubcoreMesh(
            core_axis_name="c", subcore_axis_name="s",
            num_cores=NC, num_subcores=NS),
        compiler_params=pltpu.CompilerParams(use_tc_tiling_on_sc=True,
                                             needs_layout_passes=False),
        scratch_types=(
            [pltpu.VMEM((C,), jnp.int32),
             pltpu.VMEM((C,), jnp.int32),
             pltpu.VMEM((C + L + 1,), jnp.int32),
             pltpu.VMEM((C + 1, W), jnp.float32)] * 2
            + [pltpu.VMEM((SEQ * HIDDEN,), jnp.float32)]
            + [pltpu.SemaphoreType.DMA] * 4
        ),
    )(xf, tblp, posf)


def kernel(x, item_table, pos_table):
    xf = x.reshape(N)
    tblp = jnp.pad(item_table, ((0, 0), (0, W - HIDDEN)))
    posf = pos_table.reshape(SEQ * HIDDEN)
    emb, mask = _sc_embed(xf, tblp, posf)
    return (emb[:, :HIDDEN].reshape(BATCH, SEQ, HIDDEN),
            mask.reshape(BATCH, SEQ).astype(bool))
